# baseline (speedup 1.0000x reference)
import functools

import jax
import jax.numpy as jnp
from jax.experimental import pallas as pl
from jax.experimental.pallas import tpu as pltpu

_EPS = 1e-16
_SEL0 = 64
_SEL1 = 128
_D = _SEL0 + _SEL1
_RBS = 128
_RS = 8
_N = 1024


def _build_masks():
    i = jax.lax.broadcasted_iota(jnp.int32, (1, _N), 1)
    bit_clear = {}
    keep_min = {}
    stages = []
    s = 2
    while s <= _SEL1:
        d = s // 2
        while d >= 1:
            stages.append((d, s))
            d //= 2
        s *= 2
    for lvl in range(3):
        d = _SEL1 // 2
        while d >= 1:
            stages.append((d, _SEL1 << (lvl + 1)))
            d //= 2
    for d, s in stages:
        if d not in bit_clear:
            bit_clear[d] = (i & d) == 0
        if (d, s) not in keep_min:
            bd = (i & d) != 0
            dirb = (i & s) != 0
            keep_min[(d, s)] = bd == dirb
    return bit_clear, keep_min, stages


def _exchange(x, d, bit_clear, keep_min):
    r = pltpu.roll(x, _N - d, 1)
    l = pltpu.roll(x, d, 1)
    p = jnp.where(bit_clear, r, l)
    lo = jnp.minimum(x, p)
    hi = jnp.maximum(x, p)
    return jnp.where(keep_min, lo, hi)


def _topk128(x, bit_clear, keep_min, stages):
    n_p1 = sum(1 for _ in ())
    idx = 0
    s = 2
    while s <= _SEL1:
        d = s // 2
        while d >= 1:
            x = _exchange(x, d, bit_clear[d], keep_min[(d, s)])
            idx += 1
            d //= 2
        s *= 2
    for lvl in range(3):
        D = _SEL1 << lvl
        x = jnp.minimum(x, pltpu.roll(x, _N - D, 1))
        sdir = _SEL1 << (lvl + 1)
        d = _SEL1 // 2
        while d >= 1:
            x = _exchange(x, d, bit_clear[d], keep_min[(d, sdir)])
            d //= 2
    return x


def _fused_kernel(xi_ref, xt_ref, tf_ref, tc_ref, box_ref,
                  w00, b00, w10, b10, w20, b20,
                  w01, b01, w11, b11, w21, b21,
                  out_ref, desc_ref):
    rb = pl.program_id(1)
    box = box_ref[0, 0]
    inv_box = 1.0 / box
    inf = jnp.float32(jnp.inf)
    bit_clear, keep_min, stages = _build_masks()
    is0 = tf_ref[0] == 0.0

    def body(c, carry):
        xi = xi_ref[0, pl.ds(c * _RS, _RS), :]
        acc = jnp.zeros((_RS, _N), jnp.float32)
        for k in range(3):
            a = xi[:, k : k + 1]
            bj = xt_ref[0, k : k + 1, :]
            t = a - bj + _EPS
            t = t - box * jnp.round(t * inv_box)
            acc = acc + t * t
        row_i = (jax.lax.broadcasted_iota(jnp.int32, (_RS, _N), 0)
                 + rb * _RBS + c * _RS)
        col_j = jax.lax.broadcasted_iota(jnp.int32, (_RS, _N), 1)
        diag = row_i == col_j
        k0 = jnp.where(diag | (~is0), inf, acc)
        k1 = jnp.where(diag | is0, inf, acc)
        x = jnp.concatenate([k0, k1], axis=0)
        x = _topk128(x, bit_clear, keep_min, stages)
        d0 = 1.0 / (jnp.sqrt(x[0:_RS, 0:_SEL0]) + _EPS)
        d1 = 1.0 / (jnp.sqrt(x[_RS : 2 * _RS, 0:_SEL1]) + _EPS)
        desc_ref[pl.ds(c * _RS, _RS), 0:_SEL0] = d0
        desc_ref[pl.ds(c * _RS, _RS), _SEL0:_D] = d1
        return carry

    jax.lax.fori_loop(0, _RBS // _RS, body, 0)

    x = desc_ref[...]

    def mlp(x, w0, b0, w1, b1, w2, b2):
        h = jnp.tanh(jnp.dot(x, w0[...], preferred_element_type=jnp.float32) + b0[...])
        h = jnp.tanh(jnp.dot(h, w1[...], preferred_element_type=jnp.float32) + b1[...])
        return jnp.dot(h, w2[...], preferred_element_type=jnp.float32) + b2[...]

    e0 = mlp(x, w00, b00, w10, b10, w20, b20)
    e1 = mlp(x, w01, b01, w11, b11, w21, b21)
    t = tc_ref[0]
    e = jnp.where(t == 0.0, e0, e1)
    part = jnp.broadcast_to(jnp.sum(e), (128,))

    @pl.when(pl.program_id(1) == 0)
    def _():
        out_ref[0, 0, :] = part

    @pl.when(pl.program_id(1) != 0)
    def _():
        out_ref[0, 0, :] = out_ref[0, 0, :] + part


def kernel(xyz, box_size, W0_t0, b0_t0, W1_t0, b1_t0, W2_t0, b2_t0,
           W0_t1, b0_t1, W1_t1, b1_t1, W2_t1, b2_t1, atomtypes):
    B, N, _ = xyz.shape
    xt = jnp.transpose(xyz, (0, 2, 1))
    tf = atomtypes.astype(jnp.float32).reshape(B, 1, N)
    tc = atomtypes.astype(jnp.float32).reshape(B, N, 1)
    box2 = box_size.reshape(1, 1)

    wargs = (W0_t0, b0_t0, W1_t0, b1_t0, W2_t0, b2_t0,
             W0_t1, b0_t1, W1_t1, b1_t1, W2_t1, b2_t1)

    out = pl.pallas_call(
        _fused_kernel,
        grid=(B, N // _RBS),
        in_specs=[
            pl.BlockSpec((1, _RBS, 3), lambda b, r: (b, r, 0)),
            pl.BlockSpec((1, 3, N), lambda b, r: (b, 0, 0)),
            pl.BlockSpec((1, 1, N), lambda b, r: (b, 0, 0)),
            pl.BlockSpec((1, _RBS, 1), lambda b, r: (b, r, 0)),
            pl.BlockSpec((1, 1), lambda b, r: (0, 0)),
        ] + [pl.BlockSpec(w.shape, functools.partial(
                 lambda nd, b, r: (0,) * nd, len(w.shape)))
             for w in wargs],
        out_specs=pl.BlockSpec((1, 1, 128), lambda b, r: (b, 0, 0)),
        out_shape=jax.ShapeDtypeStruct((B, 1, 128), jnp.float32),
        scratch_shapes=[pltpu.VMEM((_RBS, _D), jnp.float32)],
        compiler_params=pltpu.CompilerParams(
            dimension_semantics=("parallel", "arbitrary"),
        ),
    )(xyz, xt, tf, tc, box2, *wargs)

    return out[:, 0, 0]


# speedup vs baseline: 1.3013x; 1.3013x over previous
import functools

import jax
import jax.numpy as jnp
from jax.experimental import pallas as pl
from jax.experimental.pallas import tpu as pltpu

_EPS = 1e-16
_SEL0 = 64
_SEL1 = 128
_D = _SEL0 + _SEL1
_RBS = 128
_RS = 16
_N = 1024


def _build_masks():
    i = jax.lax.broadcasted_iota(jnp.int32, (1, _N), 1)

    def bit(b):
        return (i & b) != 0

    bit_clear = {d: (i & d) == 0 for d in (1, 2, 4, 8, 16, 32, 64)}
    pm = lambda m: jnp.where(m, jnp.float32(-1.0), jnp.float32(1.0))
    sign0 = pm(bit(2))
    phi = {s: pm(bit(s) ^ bit(2 * s)) for s in (2, 4, 8, 16, 32, 64)}
    psi = {lvl: pm(bit(_SEL1 << (lvl + 1))) for lvl in range(3)}
    return bit_clear, sign0, phi, psi


def _exchange_asc(x, d, bit_clear):
    r = pltpu.roll(x, _N - d, 1)
    l = pltpu.roll(x, d, 1)
    return jnp.where(bit_clear[d], jnp.minimum(x, r), jnp.maximum(x, l))


def _topk128(x, bit_clear, sign0, phi, psi):
    x = x * sign0
    s = 2
    while s <= _SEL1:
        d = s // 2
        while d >= 1:
            x = _exchange_asc(x, d, bit_clear)
            d //= 2
        if s < _SEL1:
            x = x * phi[s]
        s *= 2
    for lvl in range(3):
        D = _SEL1 << lvl
        r = pltpu.roll(x, _N - D, 1)
        x = jnp.minimum(x, -r)
        x = x * psi[lvl]
        d = _SEL1 // 2
        while d >= 1:
            x = _exchange_asc(x, d, bit_clear)
            d //= 2
    return x


def _fused_kernel(xi_ref, xt_ref, tf_ref, tc_ref, box_ref,
                  w00, b00, w10, b10, w20, b20,
                  w01, b01, w11, b11, w21, b21,
                  out_ref, desc_ref):
    rb = pl.program_id(1)
    box = box_ref[0, 0]
    inv_box = 1.0 / box
    inf = jnp.float32(jnp.inf)
    bit_clear, sign0, phi, psi = _build_masks()
    is0 = tf_ref[0] == 0.0

    def body(c, carry):
        xi = xi_ref[0, pl.ds(c * _RS, _RS), :]
        acc = jnp.zeros((_RS, _N), jnp.float32)
        for k in range(3):
            a = xi[:, k : k + 1]
            bj = xt_ref[0, k : k + 1, :]
            t = a - bj + _EPS
            t = t - box * jnp.round(t * inv_box)
            acc = acc + t * t
        row_i = (jax.lax.broadcasted_iota(jnp.int32, (_RS, _N), 0)
                 + rb * _RBS + c * _RS)
        col_j = jax.lax.broadcasted_iota(jnp.int32, (_RS, _N), 1)
        diag = row_i == col_j
        k0 = jnp.where(diag | (~is0), inf, acc)
        k1 = jnp.where(diag | is0, inf, acc)
        x = jnp.concatenate([k0, k1], axis=0)
        x = _topk128(x, bit_clear, sign0, phi, psi)
        d0 = 1.0 / (jnp.sqrt(x[0:_RS, 0:_SEL0]) + _EPS)
        d1 = 1.0 / (jnp.sqrt(x[_RS : 2 * _RS, 0:_SEL1]) + _EPS)
        desc_ref[pl.ds(c * _RS, _RS), 0:_SEL0] = d0
        desc_ref[pl.ds(c * _RS, _RS), _SEL0:_D] = d1
        return carry

    jax.lax.fori_loop(0, _RBS // _RS, body, 0)

    x = desc_ref[...]

    def mlp(x, w0, b0, w1, b1, w2, b2):
        h = jnp.tanh(jnp.dot(x, w0[...], preferred_element_type=jnp.float32) + b0[...])
        h = jnp.tanh(jnp.dot(h, w1[...], preferred_element_type=jnp.float32) + b1[...])
        return jnp.dot(h, w2[...], preferred_element_type=jnp.float32) + b2[...]

    e0 = mlp(x, w00, b00, w10, b10, w20, b20)
    e1 = mlp(x, w01, b01, w11, b11, w21, b21)
    t = tc_ref[0]
    e = jnp.where(t == 0.0, e0, e1)
    part = jnp.broadcast_to(jnp.sum(e), (128,))

    @pl.when(pl.program_id(1) == 0)
    def _():
        out_ref[0, 0, :] = part

    @pl.when(pl.program_id(1) != 0)
    def _():
        out_ref[0, 0, :] = out_ref[0, 0, :] + part


def kernel(xyz, box_size, W0_t0, b0_t0, W1_t0, b1_t0, W2_t0, b2_t0,
           W0_t1, b0_t1, W1_t1, b1_t1, W2_t1, b2_t1, atomtypes):
    B, N, _ = xyz.shape
    xt = jnp.transpose(xyz, (0, 2, 1))
    tf = atomtypes.astype(jnp.float32).reshape(B, 1, N)
    tc = atomtypes.astype(jnp.float32).reshape(B, N, 1)
    box2 = box_size.reshape(1, 1)

    wargs = (W0_t0, b0_t0, W1_t0, b1_t0, W2_t0, b2_t0,
             W0_t1, b0_t1, W1_t1, b1_t1, W2_t1, b2_t1)

    out = pl.pallas_call(
        _fused_kernel,
        grid=(B, N // _RBS),
        in_specs=[
            pl.BlockSpec((1, _RBS, 3), lambda b, r: (b, r, 0)),
            pl.BlockSpec((1, 3, N), lambda b, r: (b, 0, 0)),
            pl.BlockSpec((1, 1, N), lambda b, r: (b, 0, 0)),
            pl.BlockSpec((1, _RBS, 1), lambda b, r: (b, r, 0)),
            pl.BlockSpec((1, 1), lambda b, r: (0, 0)),
        ] + [pl.BlockSpec(w.shape, functools.partial(
                 lambda nd, b, r: (0,) * nd, len(w.shape)))
             for w in wargs],
        out_specs=pl.BlockSpec((1, 1, 128), lambda b, r: (b, 0, 0)),
        out_shape=jax.ShapeDtypeStruct((B, 1, 128), jnp.float32),
        scratch_shapes=[pltpu.VMEM((_RBS, _D), jnp.float32)],
        compiler_params=pltpu.CompilerParams(
            dimension_semantics=("parallel", "arbitrary"),
        ),
    )(xyz, xt, tf, tc, box2, *wargs)

    return out[:, 0, 0]


# speedup vs baseline: 6.8007x; 5.2260x over previous
import functools

import jax
import jax.numpy as jnp
from jax.experimental import pallas as pl
from jax.experimental.pallas import tpu as pltpu

_EPS = 1e-16
_SEL0 = 64
_SEL1 = 128
_D = _SEL0 + _SEL1
_K = 128
_N = 1024
_NG = _N // _K
_RL = 128


def _pm(mask):
    return jnp.where(mask, jnp.float32(-1.0), jnp.float32(1.0))


def _exch(x, d, jcol):
    k, l = x.shape
    if d >= 8:
        v = x.reshape(k // (2 * d), 2, d, l)
        lo = jnp.minimum(v[:, 0], v[:, 1])
        hi = jnp.maximum(v[:, 0], v[:, 1])
        return jnp.stack([lo, hi], axis=1).reshape(k, l)
    r = pltpu.roll(x, k - d, 0)
    lft = pltpu.roll(x, d, 0)
    bc = (jcol & d) == 0
    return jnp.where(bc, jnp.minimum(x, r), jnp.maximum(x, lft))


def _sort_group(x, gpar, jcol):
    x = x * _pm((jcol & 2) != 0)
    s = 2
    while s <= _K:
        d = s // 2
        while d >= 1:
            x = _exch(x, d, jcol)
            d //= 2
        if s < _K:
            if 2 * s < _K:
                x = x * _pm(((jcol & s) != 0) ^ ((jcol & 2 * s) != 0))
            else:
                x = x * (_pm((jcol & s) != 0) * gpar)
        s *= 2
    return x


def _fused_kernel(xyzc_ref, xtr_ref, tfc_ref, tcr_ref, box_ref,
                  w00, b00, w10, b10, w20, b20,
                  w01, b01, w11, b11, w21, b21,
                  out_ref, xs_ref):
    r0 = pl.program_id(1)
    box = box_ref[0, 0]
    inv_box = 1.0 / box
    inf = jnp.float32(jnp.inf)
    jcol = jax.lax.broadcasted_iota(jnp.int32, (_K, 1), 0)
    lane_i = jax.lax.broadcasted_iota(jnp.int32, (1, _RL), 1) + r0 * _RL

    def body(g, carry):
        cs = pl.ds(g * _K, _K)
        xc = xyzc_ref[0, cs, :]
        acc = jnp.zeros((_K, _RL), jnp.float32)
        for k in range(3):
            t = xc[:, k : k + 1] - xtr_ref[0, k : k + 1, :] + _EPS
            t = t - box * jnp.round(t * inv_box)
            acc = acc + t * t
        cand_i = jcol + g * _K
        diag = cand_i == lane_i
        tj = tfc_ref[0, cs, :]
        m0 = jnp.where(diag | (tj != 0.0), inf, acc)
        m1 = jnp.where(diag | (tj == 0.0), inf, acc)
        blk = jnp.concatenate([m0, m1], axis=1)
        gpar = jnp.where(g % 2 == 0, jnp.float32(1.0), jnp.float32(-1.0))
        xs_ref[cs, :] = _sort_group(blk, gpar, jcol)
        return carry

    jax.lax.fori_loop(0, _NG, body, 0)

    L = 2 * _RL
    groups = [xs_ref[pl.ds(p * _K, _K), :] for p in range(_NG)]
    while len(groups) > 1:
        merged = [jnp.minimum(groups[2 * p], -groups[2 * p + 1])
                  for p in range(len(groups) // 2)]
        m = jnp.concatenate(merged, axis=1)
        if len(merged) > 1:
            lanes = jax.lax.broadcasted_iota(jnp.int32, (1, m.shape[1]), 1)
            m = m * _pm((lanes & L) != 0)
        d = _K // 2
        while d >= 1:
            m = _exch(m, d, jcol)
            d //= 2
        groups = [m[:, p * L : (p + 1) * L] for p in range(len(merged))]
    x = groups[0]

    d0 = 1.0 / (jnp.sqrt(x[0:_SEL0, 0:_RL]) + _EPS)
    d1 = 1.0 / (jnp.sqrt(x[0:_SEL1, _RL : 2 * _RL]) + _EPS)
    descT = jnp.concatenate([d0, d1], axis=0)

    def mlp(w0, b0, w1, b1, w2, b2):
        h = jax.lax.dot_general(descT, w0[...], (((0,), (0,)), ((), ())),
                                preferred_element_type=jnp.float32)
        h = jnp.tanh(h + b0[...])
        h = jnp.tanh(jnp.dot(h, w1[...], preferred_element_type=jnp.float32)
                     + b1[...])
        return jnp.dot(h, w2[...], preferred_element_type=jnp.float32) + b2[...]

    e0 = mlp(w00, b00, w10, b10, w20, b20)
    e1 = mlp(w01, b01, w11, b11, w21, b21)
    t = tcr_ref[0]
    e = jnp.where(t == 0.0, e0, e1)
    part = jnp.broadcast_to(jnp.sum(e), (128,))

    @pl.when(r0 == 0)
    def _():
        out_ref[0, 0, :] = part

    @pl.when(r0 != 0)
    def _():
        out_ref[0, 0, :] = out_ref[0, 0, :] + part


def kernel(xyz, box_size, W0_t0, b0_t0, W1_t0, b1_t0, W2_t0, b2_t0,
           W0_t1, b0_t1, W1_t1, b1_t1, W2_t1, b2_t1, atomtypes):
    B, N, _ = xyz.shape
    xt = jnp.transpose(xyz, (0, 2, 1))
    tcol = atomtypes.astype(jnp.float32).reshape(B, N, 1)
    box2 = box_size.reshape(1, 1)

    wargs = (W0_t0, b0_t0, W1_t0, b1_t0, W2_t0, b2_t0,
             W0_t1, b0_t1, W1_t1, b1_t1, W2_t1, b2_t1)

    out = pl.pallas_call(
        _fused_kernel,
        grid=(B, N // _RL),
        in_specs=[
            pl.BlockSpec((1, N, 3), lambda b, r: (b, 0, 0)),
            pl.BlockSpec((1, 3, _RL), lambda b, r: (b, 0, r)),
            pl.BlockSpec((1, N, 1), lambda b, r: (b, 0, 0)),
            pl.BlockSpec((1, _RL, 1), lambda b, r: (b, r, 0)),
            pl.BlockSpec((1, 1), lambda b, r: (0, 0)),
        ] + [pl.BlockSpec(w.shape, functools.partial(
                 lambda nd, b, r: (0,) * nd, len(w.shape)))
             for w in wargs],
        out_specs=pl.BlockSpec((1, 1, 128), lambda b, r: (b, 0, 0)),
        out_shape=jax.ShapeDtypeStruct((B, 1, 128), jnp.float32),
        scratch_shapes=[pltpu.VMEM((_N, 2 * _RL), jnp.float32)],
        compiler_params=pltpu.CompilerParams(
            dimension_semantics=("parallel", "arbitrary"),
        ),
    )(xyz, xt, tcol, tcol, box2, *wargs)

    return out[:, 0, 0]


# speedup vs baseline: 6.9454x; 1.0213x over previous
import functools

import jax
import jax.numpy as jnp
from jax.experimental import pallas as pl
from jax.experimental.pallas import tpu as pltpu

_EPS = 1e-16
_SEL0 = 64
_SEL1 = 128
_D = _SEL0 + _SEL1
_K = 128
_N = 1024
_NG = _N // _K
_RL = 128


def _pm(mask):
    return jnp.where(mask, jnp.float32(-1.0), jnp.float32(1.0))


def _consts(jcol):
    bc = {d: (jcol & d) == 0 for d in (1, 2, 4)}
    sign0 = _pm((jcol & 2) != 0)
    phi = {s: _pm(((jcol & s) != 0) ^ ((jcol & 2 * s) != 0))
           for s in (2, 4, 8, 16, 32)}
    phi_last = _pm((jcol & (_K // 2)) != 0)
    return bc, sign0, phi, phi_last


def _exch(x, d, bc):
    k, l = x.shape
    if d >= 8:
        v = x.reshape(k // (2 * d), 2, d, l)
        lo = jnp.minimum(v[:, 0], v[:, 1])
        hi = jnp.maximum(v[:, 0], v[:, 1])
        return jnp.stack([lo, hi], axis=1).reshape(k, l)
    r = pltpu.roll(x, k - d, 0)
    lft = pltpu.roll(x, d, 0)
    return jnp.where(bc[d], jnp.minimum(x, r), jnp.maximum(x, lft))


def _sort_group(x, gpar, bc, sign0, phi, phi_last):
    x = x * sign0
    s = 2
    while s <= _K:
        d = s // 2
        while d >= 1:
            x = _exch(x, d, bc)
            d //= 2
        if s < _K:
            if 2 * s < _K:
                x = x * phi[s]
            else:
                x = x * (phi_last * gpar)
        s *= 2
    return x


def _fused_kernel(xyzc_ref, xtr_ref, tfc_ref, tcr_ref, box_ref,
                  w00, b00, w10, b10, w20, b20,
                  w01, b01, w11, b11, w21, b21,
                  out_ref, xs_ref):
    r0 = pl.program_id(1)
    box = box_ref[0, 0]
    inv_box = 1.0 / box
    inf = jnp.float32(jnp.inf)
    jcol = jax.lax.broadcasted_iota(jnp.int32, (_K, 1), 0)
    lane_i = jax.lax.broadcasted_iota(jnp.int32, (1, _RL), 1) + r0 * _RL
    bc, sign0, phi, phi_last = _consts(jcol)

    def body(g, carry):
        cs = pl.ds(g * _K, _K)
        xc = xyzc_ref[0, cs, :]
        acc = jnp.zeros((_K, _RL), jnp.float32)
        for k in range(3):
            t = xc[:, k : k + 1] - xtr_ref[0, k : k + 1, :] + _EPS
            t = t - box * jnp.round(t * inv_box)
            acc = acc + t * t
        cand_i = jcol + g * _K
        diag = cand_i == lane_i
        tj = tfc_ref[0, cs, :]
        m0 = jnp.where(diag | (tj != 0.0), inf, acc)
        m1 = jnp.where(diag | (tj == 0.0), inf, acc)
        blk = jnp.concatenate([m0, m1], axis=1)
        gpar = jnp.where(g % 2 == 0, jnp.float32(1.0), jnp.float32(-1.0))
        xs_ref[cs, :] = _sort_group(blk, gpar, bc, sign0, phi, phi_last)
        return carry

    jax.lax.fori_loop(0, _NG, body, 0)

    L = 2 * _RL
    groups = [xs_ref[pl.ds(p * _K, _K), :] for p in range(_NG)]
    while len(groups) > 1:
        merged = [jnp.minimum(groups[2 * p], -groups[2 * p + 1])
                  for p in range(len(groups) // 2)]
        m = jnp.concatenate(merged, axis=1)
        if len(merged) > 1:
            lanes = jax.lax.broadcasted_iota(jnp.int32, (1, m.shape[1]), 1)
            m = m * _pm((lanes & L) != 0)
        d = _K // 2
        while d >= 1:
            m = _exch(m, d, bc)
            d //= 2
        groups = [m[:, p * L : (p + 1) * L] for p in range(len(merged))]
    x = groups[0]

    d0 = 1.0 / (jnp.sqrt(x[0:_SEL0, 0:_RL]) + _EPS)
    d1 = 1.0 / (jnp.sqrt(x[0:_SEL1, _RL : 2 * _RL]) + _EPS)
    descT = jnp.concatenate([d0, d1], axis=0)

    def mlp(w0, b0, w1, b1, w2, b2):
        h = jax.lax.dot_general(descT, w0[...], (((0,), (0,)), ((), ())),
                                preferred_element_type=jnp.float32)
        h = jnp.tanh(h + b0[...])
        h = jnp.tanh(jnp.dot(h, w1[...], preferred_element_type=jnp.float32)
                     + b1[...])
        return jnp.dot(h, w2[...], preferred_element_type=jnp.float32) + b2[...]

    e0 = mlp(w00, b00, w10, b10, w20, b20)
    e1 = mlp(w01, b01, w11, b11, w21, b21)
    t = tcr_ref[0]
    e = jnp.where(t == 0.0, e0, e1)
    part = jnp.broadcast_to(jnp.sum(e), (128,))

    @pl.when(r0 == 0)
    def _():
        out_ref[0, 0, :] = part

    @pl.when(r0 != 0)
    def _():
        out_ref[0, 0, :] = out_ref[0, 0, :] + part


def kernel(xyz, box_size, W0_t0, b0_t0, W1_t0, b1_t0, W2_t0, b2_t0,
           W0_t1, b0_t1, W1_t1, b1_t1, W2_t1, b2_t1, atomtypes):
    B, N, _ = xyz.shape
    xt = jnp.transpose(xyz, (0, 2, 1))
    tcol = atomtypes.astype(jnp.float32).reshape(B, N, 1)
    box2 = box_size.reshape(1, 1)

    wargs = (W0_t0, b0_t0, W1_t0, b1_t0, W2_t0, b2_t0,
             W0_t1, b0_t1, W1_t1, b1_t1, W2_t1, b2_t1)

    out = pl.pallas_call(
        _fused_kernel,
        grid=(B, N // _RL),
        in_specs=[
            pl.BlockSpec((1, N, 3), lambda b, r: (b, 0, 0)),
            pl.BlockSpec((1, 3, _RL), lambda b, r: (b, 0, r)),
            pl.BlockSpec((1, N, 1), lambda b, r: (b, 0, 0)),
            pl.BlockSpec((1, _RL, 1), lambda b, r: (b, r, 0)),
            pl.BlockSpec((1, 1), lambda b, r: (0, 0)),
        ] + [pl.BlockSpec(w.shape, functools.partial(
                 lambda nd, b, r: (0,) * nd, len(w.shape)))
             for w in wargs],
        out_specs=pl.BlockSpec((1, 1, 128), lambda b, r: (b, 0, 0)),
        out_shape=jax.ShapeDtypeStruct((B, 1, 128), jnp.float32),
        scratch_shapes=[pltpu.VMEM((_N, 2 * _RL), jnp.float32)],
        compiler_params=pltpu.CompilerParams(
            dimension_semantics=("parallel", "arbitrary"),
        ),
    )(xyz, xt, tcol, tcol, box2, *wargs)

    return out[:, 0, 0]


# speedup vs baseline: 10.1152x; 1.4564x over previous
import functools

import jax
import jax.numpy as jnp
from jax import lax
from jax.experimental import pallas as pl
from jax.experimental.pallas import tpu as pltpu
from jax.experimental.pallas import tpu_sc as plsc

_EPS = 1e-16
_SEL0 = 64
_SEL1 = 128
_D = _SEL0 + _SEL1
_N = 1024
_K = 128
_NREG = _K // 16
_BLK = 16
_RB = 256


def _dist_kernel(xi_ref, xt_ref, tf_ref, box_ref, out_ref):
    rb = pl.program_id(1)
    box = box_ref[0, 0]
    inv_box = 1.0 / box
    xi = xi_ref[0]
    acc = jnp.zeros((_RB, _N), jnp.float32)
    for k in range(3):
        t = xi[:, k : k + 1] - xt_ref[0, k : k + 1, :] + _EPS
        t = t - box * jnp.round(t * inv_box)
        acc = acc + t * t
    row_i = jax.lax.broadcasted_iota(jnp.int32, (_RB, _N), 0) + rb * _RB
    col_j = jax.lax.broadcasted_iota(jnp.int32, (_RB, _N), 1)
    diag = row_i == col_j
    is0 = tf_ref[0] == 0.0
    inf = jnp.float32(jnp.inf)
    out_ref[0, 0] = jnp.where(diag | (~is0), inf, acc)
    out_ref[0, 1] = jnp.where(diag | is0, inf, acc)


def _s16(v):
    r = plsc.sort_key_val(v, v)
    return r[0] if isinstance(r, (tuple, list)) else r


def _bitonic_fix(regs):
    regs = list(regs)
    n = len(regs)
    d = n // 2
    while d >= 1:
        for base in range(0, n, 2 * d):
            for off in range(d):
                a = regs[base + off]
                b = regs[base + off + d]
                regs[base + off] = jnp.minimum(a, b)
                regs[base + off + d] = jnp.maximum(a, b)
        d //= 2
    return [_s16(r) for r in regs]


def _merge(a, b):
    m = len(a)
    fb = [jnp.flip(b[m - 1 - i], 0) for i in range(m)]
    lo = [jnp.minimum(a[i], fb[i]) for i in range(m)]
    hi = [jnp.maximum(a[i], fb[i]) for i in range(m)]
    return _bitonic_fix(lo), _bitonic_fix(hi)


def _sort_chunk(vs):
    rs = [_s16(v) for v in vs]
    width = 1
    while width < len(rs):
        nxt = []
        for p in range(0, len(rs), 2 * width):
            lo, hi = _merge(rs[p : p + width], rs[p + width : p + 2 * width])
            nxt.extend(lo + hi)
        rs = nxt
        width *= 2
    return rs


def _merge_keep_lo(buf, c):
    m = len(buf)
    fc = [jnp.flip(c[m - 1 - i], 0) for i in range(m)]
    lo = [jnp.minimum(buf[i], fc[i]) for i in range(m)]
    return _bitonic_fix(lo)


def _make_sc_topk(R):
    mesh = plsc.VectorSubcoreMesh(core_axis_name="c", subcore_axis_name="s")
    rpw = R // 32
    nblk = rpw // _BLK

    @functools.partial(
        pl.kernel, mesh=mesh,
        compiler_params=pltpu.CompilerParams(needs_layout_passes=False),
        out_type=jax.ShapeDtypeStruct((R * _K,), jnp.float32),
        scratch_types=[
            pltpu.VMEM((_BLK * _N,), jnp.float32),
            pltpu.VMEM((_BLK * _K,), jnp.float32),
        ],
    )
    def k(x_hbm, out_hbm, buf_v, out_v):
        wid = lax.axis_index("s") * 2 + lax.axis_index("c")
        base_row = wid * rpw

        def blk_body(bi, _):
            row0 = base_row + bi * _BLK
            pltpu.sync_copy(x_hbm.at[pl.ds(row0 * _N, _BLK * _N)], buf_v)

            def row_body(j, _):
                def vreg(t):
                    return buf_v[pl.ds(j * _N + t * 16, 16)]

                buf = _sort_chunk([vreg(t) for t in range(8)])
                for c in range(1, 8):
                    ch = _sort_chunk([vreg(8 * c + t) for t in range(8)])
                    buf = _merge_keep_lo(buf, ch)
                for r in range(_NREG):
                    out_v[pl.ds(j * _K + r * 16, 16)] = buf[r]
                return 0

            lax.fori_loop(0, _BLK, row_body, 0)
            pltpu.sync_copy(out_v, out_hbm.at[pl.ds(row0 * _K, _BLK * _K)])
            return 0

        lax.fori_loop(0, nblk, blk_body, 0)

    return k


def _mlp_kernel(sq_ref, tc_ref,
                w00, b00, w10, b10, w20, b20,
                w01, b01, w11, b11, w21, b21,
                out_ref):
    s0 = sq_ref[0, 0, :, 0:_SEL0]
    s1 = sq_ref[0, 1, :, 0:_SEL1]
    desc = jnp.concatenate(
        [1.0 / (jnp.sqrt(s0) + _EPS), 1.0 / (jnp.sqrt(s1) + _EPS)], axis=1)

    def mlp(x, w0, b0, w1, b1, w2, b2):
        h = jnp.tanh(jnp.dot(x, w0[...], preferred_element_type=jnp.float32) + b0[...])
        h = jnp.tanh(jnp.dot(h, w1[...], preferred_element_type=jnp.float32) + b1[...])
        return jnp.dot(h, w2[...], preferred_element_type=jnp.float32) + b2[...]

    e0 = mlp(desc, w00, b00, w10, b10, w20, b20)
    e1 = mlp(desc, w01, b01, w11, b11, w21, b21)
    t = tc_ref[0]
    e = jnp.where(t == 0.0, e0, e1)
    out_ref[0, 0, :] = jnp.broadcast_to(jnp.sum(e), (128,))


def kernel(xyz, box_size, W0_t0, b0_t0, W1_t0, b1_t0, W2_t0, b2_t0,
           W0_t1, b0_t1, W1_t1, b1_t1, W2_t1, b2_t1, atomtypes):
    B, N, _ = xyz.shape
    xt = jnp.transpose(xyz, (0, 2, 1))
    tf = atomtypes.astype(jnp.float32).reshape(B, 1, N)
    tcol = atomtypes.astype(jnp.float32).reshape(B, N, 1)
    box2 = box_size.reshape(1, 1)

    masked = pl.pallas_call(
        _dist_kernel,
        grid=(B, N // _RB),
        in_specs=[
            pl.BlockSpec((1, _RB, 3), lambda b, r: (b, r, 0)),
            pl.BlockSpec((1, 3, N), lambda b, r: (b, 0, 0)),
            pl.BlockSpec((1, 1, N), lambda b, r: (b, 0, 0)),
            pl.BlockSpec((1, 1), lambda b, r: (0, 0)),
        ],
        out_specs=pl.BlockSpec((1, 2, _RB, N), lambda b, r: (b, 0, r, 0)),
        out_shape=jax.ShapeDtypeStruct((B, 2, N, N), jnp.float32),
        compiler_params=pltpu.CompilerParams(
            dimension_semantics=("parallel", "parallel"),
        ),
    )(xyz, xt, tf, box2)

    R = B * 2 * N
    sel = _make_sc_topk(R)(masked.reshape(R * _N))
    sq = sel.reshape(B, 2, N, _K)

    wargs = (W0_t0, b0_t0, W1_t0, b1_t0, W2_t0, b2_t0,
             W0_t1, b0_t1, W1_t1, b1_t1, W2_t1, b2_t1)
    out = pl.pallas_call(
        _mlp_kernel,
        grid=(B,),
        in_specs=[
            pl.BlockSpec((1, 2, N, _K), lambda b: (b, 0, 0, 0)),
            pl.BlockSpec((1, N, 1), lambda b: (b, 0, 0)),
        ] + [pl.BlockSpec(w.shape, functools.partial(
                 lambda nd, b: (0,) * nd, len(w.shape)))
             for w in wargs],
        out_specs=pl.BlockSpec((1, 1, 128), lambda b: (b, 0, 0)),
        out_shape=jax.ShapeDtypeStruct((B, 1, 128), jnp.float32),
    )(sq, tcol, *wargs)

    return out[:, 0, 0]
